# baseline (speedup 1.0000x reference)
import functools

import numpy as np

import jax
import jax.numpy as jnp
from jax import lax
from jax.experimental import pallas as pl
from jax.experimental.pallas import tpu as pltpu
from jax.experimental.pallas import tpu_sc as plsc

NC = 2
NS = 16
LANES = 16
NPAD = 10240
K = 16


def _proj(x, wa, wb, heads, dp, bn=1024):
    n, kd = x.shape
    nb = n // bn

    def body(x_ref, wa_ref, wb_ref, oa_ref, ob_ref):
        xv = x_ref[...]
        oa_ref[...] = jnp.dot(xv, wa_ref[...], preferred_element_type=jnp.float32)
        ob_ref[...] = jnp.dot(xv, wb_ref[...], preferred_element_type=jnp.float32)

    out_shape = jax.ShapeDtypeStruct((heads * n, dp), jnp.float32)
    return pl.pallas_call(
        body,
        grid=(heads, nb),
        in_specs=[
            pl.BlockSpec((bn, kd), lambda h, i: (i, 0)),
            pl.BlockSpec((kd, dp), lambda h, i: (0, h)),
            pl.BlockSpec((kd, dp), lambda h, i: (0, h)),
        ],
        out_specs=[
            pl.BlockSpec((bn, dp), lambda h, i: (h * nb + i, 0)),
            pl.BlockSpec((bn, dp), lambda h, i: (h * nb + i, 0)),
        ],
        out_shape=[out_shape, out_shape],
    )(x, wa, wb)


def _combine(acc, bias, heads, d, n, relu, bn=1024):
    nb = n // bn
    dw = d + LANES

    def body(a_ref, b_ref, o_ref):
        a = a_ref[...]
        num = a[0, 0, :, :d] + a[1, 0, :, :d]
        den = a[0, 0, :, d] + a[1, 0, :, d]
        r = num / (den[:, None] + 1e-16) + b_ref[...][None, :]
        if relu:
            r = jnp.maximum(r, 0.0)
        o_ref[...] = r

    return pl.pallas_call(
        body,
        grid=(heads, nb),
        in_specs=[
            pl.BlockSpec((2, 1, bn, dw), lambda h, i: (0, h, i, 0)),
            pl.BlockSpec((d,), lambda h, i: (h,)),
        ],
        out_specs=pl.BlockSpec((bn, d), lambda h, i: (i, h)),
        out_shape=jax.ShapeDtypeStruct((n, heads * d), jnp.float32),
    )(acc, bias)


@functools.lru_cache(maxsize=None)
def _edge_pass(heads, n_pad, ep, dp):
    te = ep // (NC * NS)
    nbatch = te // K
    rows_per_tile = n_pad // NS
    dj = dp // LANES
    dw = dp + LANES
    zrows = 16
    mesh = plsc.VectorSubcoreMesh(core_axis_name="c", subcore_axis_name="s",
                                  num_cores=NC, num_subcores=NS)

    @functools.partial(
        pl.kernel,
        out_type=jax.ShapeDtypeStruct((NC, heads, n_pad, dw), jnp.float32),
        mesh=mesh,
        compiler_params=pltpu.CompilerParams(use_tc_tiling_on_sc=False),
        scratch_types=[
            pltpu.VMEM_SHARED((n_pad, dw), jnp.float32),
            pltpu.VMEM((te,), jnp.int32),
            pltpu.VMEM((te,), jnp.int32),
            pltpu.VMEM((2, K, dp), jnp.float32),
            pltpu.VMEM((2, K, dp), jnp.float32),
            pltpu.VMEM((K, dw), jnp.float32),
            pltpu.VMEM((zrows, dw), jnp.float32),
            pltpu.VMEM((dp,), jnp.float32),
            pltpu.VMEM((LANES,), jnp.int32),
            pltpu.SemaphoreType.DMA((2,)),
        ],
    )
    def edge_kernel(xl_hbm, xr_hbm, src_hbm, dst_hbm, att_hbm, iv_hbm, out_hbm,
                    accum, src_v, dst_v, xl_b, xr_b, stg, zbuf, att_v, iv_v,
                    sems):
        c = lax.axis_index("c")
        s = lax.axis_index("s")
        wid = c * NS + s
        e0 = pl.multiple_of(wid * te, 8)
        pltpu.sync_copy(src_hbm.at[pl.ds(e0, te)], src_v)
        pltpu.sync_copy(dst_hbm.at[pl.ds(e0, te)], dst_v)

        pltpu.sync_copy(iv_hbm, iv_v)
        ix = iv_v[...]
        zv = ix.astype(jnp.float32) * 0.0

        def zb_row(i, carry):
            for j in range(dw // LANES):
                zbuf[i, pl.ds(j * LANES, LANES)] = zv
            return carry
        lax.fori_loop(0, zrows, zb_row, 0)

        r0 = pl.multiple_of(s * rows_per_tile, 8)

        for h in range(heads):
            def zchunk(i, carry):
                pltpu.sync_copy(zbuf, accum.at[pl.ds(r0 + i * zrows, zrows)])
                return carry
            lax.fori_loop(0, rows_per_tile // zrows, zchunk, 0)
            plsc.subcore_barrier()

            pltpu.sync_copy(att_hbm.at[h], att_v)
            attv = [att_v[pl.ds(j * LANES, LANES)] for j in range(dj)]
            hoff = h * n_pad

            def gather(g, slot):
                gi = src_v[pl.ds(g * K, K)] + hoff
                ri = dst_v[pl.ds(g * K, K)] + hoff
                pltpu.async_copy(xl_hbm.at[gi], xl_b.at[slot], sems.at[slot])
                pltpu.async_copy(xr_hbm.at[ri], xr_b.at[slot], sems.at[slot])

            def wait(g, slot):
                gi = src_v[pl.ds(g * K, K)] + hoff
                ri = dst_v[pl.ds(g * K, K)] + hoff
                pltpu.make_async_copy(xl_hbm.at[gi], xl_b.at[slot],
                                      sems.at[slot]).wait()
                pltpu.make_async_copy(xr_hbm.at[ri], xr_b.at[slot],
                                      sems.at[slot]).wait()

            def compute(g, slot):
                def per_edge(e, carry):
                    xs = [xl_b[slot, e, pl.ds(j * LANES, LANES)] for j in range(dj)]
                    acc = zv
                    for j in range(dj):
                        t = xs[j] + xr_b[slot, e, pl.ds(j * LANES, LANES)]
                        acc = acc + jnp.maximum(t, 0.2 * t) * attv[j]
                    for sh in (1, 2, 4, 8):
                        acc = acc + acc.at[ix ^ sh].get(
                            mode="promise_in_bounds")
                    exv = jnp.exp(acc)
                    for j in range(dj):
                        stg[e, pl.ds(j * LANES, LANES)] = exv * xs[j]
                    stg[e, pl.ds(dp, LANES)] = exv
                    return carry
                def edge4(q, carry):
                    for u in range(4):
                        per_edge(q * 4 + u, 0)
                    return carry
                lax.fori_loop(0, K // 4, edge4, 0)
                di = dst_v[pl.ds(g * K, K)]
                pltpu.sync_copy(stg, accum.at[di], add=True)

            gather(0, 0)

            def pair(gg, carry):
                g0 = gg * 2
                gather(g0 + 1, 1)
                wait(g0, 0)
                compute(g0, 0)

                @pl.when(gg < nbatch // 2 - 1)
                def _():
                    gather(g0 + 2, 0)

                wait(g0 + 1, 1)
                compute(g0 + 1, 1)
                return carry
            lax.fori_loop(0, nbatch // 2, pair, 0)

            plsc.subcore_barrier()
            pltpu.sync_copy(accum.at[pl.ds(r0, rows_per_tile)],
                            out_hbm.at[c, h, pl.ds(r0, rows_per_tile)])

    return edge_kernel


def kernel(x, edge_index, Wl1, Wr1, att1, b1, Wl2, Wr2, att2, b2):
    n, in_ch = x.shape
    e = edge_index.shape[1]
    xp = jnp.zeros((NPAD, in_ch), jnp.float32).at[:n].set(x.astype(jnp.float32))
    loop = jnp.arange(n, dtype=jnp.int32)
    src = jnp.concatenate([edge_index[0].astype(jnp.int32), loop])
    dst = jnp.concatenate([edge_index[1].astype(jnp.int32), loop])
    et = e + n
    ep = -(-et // 1024) * 1024
    pad = jnp.full((ep - et,), n, jnp.int32)
    src = jnp.concatenate([src, pad])
    dst = jnp.concatenate([dst, pad])

    iv = jnp.arange(LANES, dtype=jnp.int32)
    heads1, d1 = att1.shape
    xl1, xr1 = _proj(xp, Wl1, Wr1, heads1, d1)
    acc1 = _edge_pass(heads1, NPAD, ep, d1)(xl1, xr1, src, dst, att1, iv)
    h = _combine(acc1, b1.reshape(-1), heads1, d1, NPAD, True)

    heads2, d2 = att2.shape
    xl2, xr2 = _proj(h, Wl2, Wr2, heads2, d2)
    acc2 = _edge_pass(heads2, NPAD, ep, d2)(xl2, xr2, src, dst, att2, iv)
    out = _combine(acc2, b2.reshape(-1), heads2, d2, NPAD, False)
    return out[:n]


# speedup vs baseline: 1.0030x; 1.0030x over previous
import functools

import numpy as np

import jax
import jax.numpy as jnp
from jax import lax
from jax.experimental import pallas as pl
from jax.experimental.pallas import tpu as pltpu
from jax.experimental.pallas import tpu_sc as plsc

NC = 2
NS = 16
LANES = 16
NPAD = 10240
K = 16


def _proj(x, wa, wb, heads, dp, bn=1024):
    n, kd = x.shape
    nb = n // bn

    def body(x_ref, wa_ref, wb_ref, oa_ref, ob_ref):
        xv = x_ref[...]
        oa_ref[...] = jnp.dot(xv, wa_ref[...],
                              preferred_element_type=jnp.float32)
        ob_ref[...] = jnp.dot(xv, wb_ref[...],
                              preferred_element_type=jnp.float32)

    out_shape = jax.ShapeDtypeStruct((heads * n, dp), jnp.float32)
    return pl.pallas_call(
        body,
        grid=(heads, nb),
        in_specs=[
            pl.BlockSpec((bn, kd), lambda h, i: (i, 0)),
            pl.BlockSpec((kd, dp), lambda h, i: (0, h)),
            pl.BlockSpec((kd, dp), lambda h, i: (0, h)),
        ],
        out_specs=[
            pl.BlockSpec((bn, dp), lambda h, i: (h * nb + i, 0)),
            pl.BlockSpec((bn, dp), lambda h, i: (h * nb + i, 0)),
        ],
        out_shape=[out_shape, out_shape],
    )(x, wa, wb)


def _combine(acc, bias, heads, d, n, relu, bn=1024):
    nb = n // bn
    dw = d + LANES

    def body(a_ref, b_ref, o_ref):
        a = a_ref[...]
        num = a[0, 0, :, :d] + a[1, 0, :, :d]
        den = a[0, 0, :, d] + a[1, 0, :, d]
        r = num / (den[:, None] + 1e-16) + b_ref[...][None, :]
        if relu:
            r = jnp.maximum(r, 0.0)
        o_ref[...] = r

    return pl.pallas_call(
        body,
        grid=(heads, nb),
        in_specs=[
            pl.BlockSpec((2, 1, bn, dw), lambda h, i: (0, h, i, 0)),
            pl.BlockSpec((d,), lambda h, i: (h,)),
        ],
        out_specs=pl.BlockSpec((bn, d), lambda h, i: (i, h)),
        out_shape=jax.ShapeDtypeStruct((n, heads * d), jnp.float32),
    )(acc, bias)


@functools.lru_cache(maxsize=None)
def _edge_pass(heads, n_pad, ep, dp):
    te = ep // (NC * NS)
    nbatch = te // K
    rows_per_tile = n_pad // NS
    dj = dp // LANES
    dw = dp + LANES
    zrows = 16
    mesh = plsc.VectorSubcoreMesh(core_axis_name="c", subcore_axis_name="s",
                                  num_cores=NC, num_subcores=NS)

    @functools.partial(
        pl.kernel,
        out_type=jax.ShapeDtypeStruct((NC, heads, n_pad, dw), jnp.float32),
        mesh=mesh,
        compiler_params=pltpu.CompilerParams(use_tc_tiling_on_sc=False),
        scratch_types=[
            pltpu.VMEM_SHARED((n_pad, dw), jnp.float32),
            pltpu.VMEM((te,), jnp.int32),
            pltpu.VMEM((te,), jnp.int32),
            pltpu.VMEM((2, K, dp), jnp.float32),
            pltpu.VMEM((2, K, dp), jnp.float32),
            pltpu.VMEM((K, dw), jnp.float32),
            pltpu.VMEM((zrows, dw), jnp.float32),
            pltpu.VMEM((dp,), jnp.float32),
            pltpu.VMEM((LANES,), jnp.int32),
            pltpu.SemaphoreType.DMA((2,)),
        ],
    )
    def edge_kernel(xl_hbm, xr_hbm, src_hbm, dst_hbm, att_hbm, iv_hbm, out_hbm,
                    accum, src_v, dst_v, xl_b, xr_b, stg, zbuf, att_v, iv_v,
                    sems):
        c = lax.axis_index("c")
        s = lax.axis_index("s")
        wid = c * NS + s
        e0 = pl.multiple_of(wid * te, 8)
        pltpu.sync_copy(src_hbm.at[pl.ds(e0, te)], src_v)
        pltpu.sync_copy(dst_hbm.at[pl.ds(e0, te)], dst_v)

        pltpu.sync_copy(iv_hbm, iv_v)
        ix = iv_v[...]
        zv = ix.astype(jnp.float32) * 0.0

        def zb_row(i, carry):
            for j in range(dw // LANES):
                zbuf[i, pl.ds(j * LANES, LANES)] = zv
            return carry
        lax.fori_loop(0, zrows, zb_row, 0)

        r0 = pl.multiple_of(s * rows_per_tile, 8)

        for h in range(heads):
            def zchunk(i, carry):
                pltpu.sync_copy(zbuf, accum.at[pl.ds(r0 + i * zrows, zrows)])
                return carry
            lax.fori_loop(0, rows_per_tile // zrows, zchunk, 0)
            plsc.subcore_barrier()

            pltpu.sync_copy(att_hbm.at[h], att_v)
            attv = [att_v[pl.ds(j * LANES, LANES)] for j in range(dj)]
            hoff = h * n_pad

            def gather(g, slot):
                gi = src_v[pl.ds(g * K, K)] + hoff
                ri = dst_v[pl.ds(g * K, K)] + hoff
                pltpu.async_copy(xl_hbm.at[gi], xl_b.at[slot], sems.at[slot])
                pltpu.async_copy(xr_hbm.at[ri], xr_b.at[slot], sems.at[slot])

            def wait(g, slot):
                gi = src_v[pl.ds(g * K, K)] + hoff
                ri = dst_v[pl.ds(g * K, K)] + hoff
                pltpu.make_async_copy(xl_hbm.at[gi], xl_b.at[slot],
                                      sems.at[slot]).wait()
                pltpu.make_async_copy(xr_hbm.at[ri], xr_b.at[slot],
                                      sems.at[slot]).wait()

            def compute(g, slot):
                def per_edge(e, carry):
                    xs = []
                    acc = zv
                    for j in range(dj):
                        xj = xl_b[slot, e, pl.ds(j * LANES, LANES)]
                        rj = xr_b[slot, e, pl.ds(j * LANES, LANES)]
                        xs.append(xj)
                        t = xj + rj
                        acc = acc + jnp.maximum(t, 0.2 * t) * attv[j]
                    for sh in (1, 2, 4, 8):
                        acc = acc + acc.at[ix ^ sh].get(
                            mode="promise_in_bounds")
                    exv = jnp.exp(acc)
                    for j in range(dj):
                        stg[e, pl.ds(j * LANES, LANES)] = exv * xs[j]
                    stg[e, pl.ds(dp, LANES)] = exv
                    return carry
                lax.fori_loop(0, K, per_edge, 0)
                di = dst_v[pl.ds(g * K, K)]
                pltpu.sync_copy(stg, accum.at[di], add=True)

            gather(0, 0)

            def pair(gg, carry):
                g0 = gg * 2
                gather(g0 + 1, 1)
                wait(g0, 0)
                compute(g0, 0)

                @pl.when(gg < nbatch // 2 - 1)
                def _():
                    gather(g0 + 2, 0)

                wait(g0 + 1, 1)
                compute(g0 + 1, 1)
                return carry
            lax.fori_loop(0, nbatch // 2, pair, 0)

            plsc.subcore_barrier()
            pltpu.sync_copy(accum.at[pl.ds(r0, rows_per_tile)],
                            out_hbm.at[c, h, pl.ds(r0, rows_per_tile)])

    return edge_kernel


def kernel(x, edge_index, Wl1, Wr1, att1, b1, Wl2, Wr2, att2, b2):
    n, in_ch = x.shape
    e = edge_index.shape[1]
    xp = jnp.zeros((NPAD, in_ch), jnp.float32).at[:n].set(x.astype(jnp.float32))
    loop = jnp.arange(n, dtype=jnp.int32)
    src = jnp.concatenate([edge_index[0].astype(jnp.int32), loop])
    dst = jnp.concatenate([edge_index[1].astype(jnp.int32), loop])
    et = e + n
    ep = -(-et // 1024) * 1024
    pad = jnp.full((ep - et,), n, jnp.int32)
    src = jnp.concatenate([src, pad])
    dst = jnp.concatenate([dst, pad])

    iv = jnp.arange(LANES, dtype=jnp.int32)
    heads1, d1 = att1.shape
    heads2, d2 = att2.shape

    xl1, xr1 = _proj(xp, Wl1, Wr1, heads1, d1)
    acc1 = _edge_pass(heads1, NPAD, ep, d1)(xl1, xr1, src, dst, att1, iv)
    h = _combine(acc1, b1, heads1, d1, NPAD, True)

    xl2, xr2 = _proj(h, Wl2, Wr2, heads2, d2)
    acc2 = _edge_pass(heads2, NPAD, ep, d2)(xl2, xr2, src, dst, att2, iv)
    out = _combine(acc2, b2, heads2, d2, NPAD, False)
    return out[:n]
